# baseline (speedup 1.0000x reference)
import jax
import jax.numpy as jnp
from jax import lax
from jax.experimental import pallas as pl
from jax.experimental.pallas import tpu as pltpu
from jax.experimental.pallas import tpu_sc as plsc

N_OUT = 4096
N_CMD = 4
BATCH = 4096

NC = 2
NS = 16
L = 16
NW = NC * NS
GROUPS = BATCH * N_OUT
G_PER_W = GROUPS // NW
CHS = 8192
N_CHUNK = G_PER_W // CHS


def _sc_body(x_hbm, cmd_hbm, out_hbm, cmd_v,
             in0, in1, out0, out1, si0, si1, so0, so1):
    wid = lax.axis_index("s") * NC + lax.axis_index("c")
    pltpu.sync_copy(cmd_hbm, cmd_v)
    c = jnp.max(cmd_v[...])
    iv = N_CMD * lax.iota(jnp.int32, L) + c
    base = wid * G_PER_W

    in_bufs = (in0, in1)
    out_bufs = (out0, out1)
    in_sems = (si0, si1)
    out_sems = (so0, so1)

    def in_copy(i):
        return pltpu.async_copy(
            x_hbm.at[pl.ds(N_CMD * (base + i * CHS), N_CMD * CHS)],
            in_bufs[i % 2], in_sems[i % 2])

    h_in = {0: in_copy(0)}
    h_out = {}
    for i in range(N_CHUNK):
        s = i % 2
        if i + 1 < N_CHUNK:
            h_in[i + 1] = in_copy(i + 1)
        h_in[i].wait()
        if i >= 2:
            h_out[i - 2].wait()

        @plsc.parallel_loop(0, CHS // L, unroll=16)
        def inner(j, _s=s):
            v = plsc.load_gather(in_bufs[_s], [iv + N_CMD * L * j])
            out_bufs[_s][pl.ds(L * j, L)] = v

        h_out[i] = pltpu.async_copy(
            out_bufs[s], out_hbm.at[pl.ds(base + i * CHS, CHS)], out_sems[s])

    h_out[N_CHUNK - 2].wait()
    h_out[N_CHUNK - 1].wait()


@jax.jit
def kernel(x, command):
    cmd16 = jnp.broadcast_to(command.astype(jnp.int32), (L,))
    xf = x.reshape(GROUPS * N_CMD)
    mesh = plsc.VectorSubcoreMesh(core_axis_name="c", subcore_axis_name="s")
    run = pl.kernel(
        _sc_body,
        out_type=jax.ShapeDtypeStruct((GROUPS,), jnp.float32),
        mesh=mesh,
        scratch_types=[
            pltpu.VMEM((L,), jnp.int32),
            pltpu.VMEM((N_CMD * CHS,), jnp.float32),
            pltpu.VMEM((N_CMD * CHS,), jnp.float32),
            pltpu.VMEM((CHS,), jnp.float32),
            pltpu.VMEM((CHS,), jnp.float32),
            pltpu.SemaphoreType.DMA,
            pltpu.SemaphoreType.DMA,
            pltpu.SemaphoreType.DMA,
            pltpu.SemaphoreType.DMA,
        ],
        compiler_params=pltpu.CompilerParams(needs_layout_passes=False),
    )
    return run(xf, cmd16).reshape(BATCH, N_OUT)


# speedup vs baseline: 3.8971x; 3.8971x over previous
import jax
import jax.numpy as jnp
from jax.experimental import pallas as pl
from jax.experimental.pallas import tpu as pltpu

N_OUT = 4096
N_CMD = 4
BATCH = 4096

BB = 128
KCH = 512
NCH = KCH // N_CMD


def _tc_body(cmd_ref, x_ref, o_ref):
    c = cmd_ref[0]
    j = jax.lax.broadcasted_iota(jnp.int32, (KCH, NCH), 0)
    n = jax.lax.broadcasted_iota(jnp.int32, (KCH, NCH), 1)
    s = (j == N_CMD * n + c).astype(jnp.float32)
    for g in range(N_OUT * N_CMD // KCH):
        o_ref[:, g * NCH:(g + 1) * NCH] = jnp.dot(
            x_ref[:, g * KCH:(g + 1) * KCH], s,
            preferred_element_type=jnp.float32)


@jax.jit
def kernel(x, command):
    grid_spec = pltpu.PrefetchScalarGridSpec(
        num_scalar_prefetch=1,
        grid=(BATCH // BB,),
        in_specs=[pl.BlockSpec((BB, N_OUT * N_CMD), lambda i, c: (i, 0))],
        out_specs=pl.BlockSpec((BB, N_OUT), lambda i, c: (i, 0)),
    )
    return pl.pallas_call(
        _tc_body,
        grid_spec=grid_spec,
        out_shape=jax.ShapeDtypeStruct((BATCH, N_OUT), jnp.float32),
        compiler_params=pltpu.CompilerParams(
            dimension_semantics=("parallel",)),
    )(command, x)
